# baseline (speedup 1.0000x reference)
import functools

import jax
import jax.numpy as jnp
from jax import lax
from jax.experimental import pallas as pl
from jax.experimental.pallas import tpu as pltpu
from jax.experimental.pallas import tpu_sc as plsc

NC = 2
NS = 16
NW = NC * NS
L = 16

B = 16384
D = 64
BPW = B // NW
CHUNK = 128
NCHUNK = BPW // CHUNK

_mesh = plsc.VectorSubcoreMesh(core_axis_name="c", subcore_axis_name="s")


@functools.partial(
    pl.kernel,
    out_type=jax.ShapeDtypeStruct((B,), jnp.float32),
    mesh=_mesh,
    compiler_params=pltpu.CompilerParams(use_tc_tiling_on_sc=False),
    scratch_types=[
        pltpu.VMEM((NCHUNK, CHUNK), jnp.int32),
        pltpu.VMEM((NCHUNK, CHUNK), jnp.int32),
        pltpu.VMEM((BPW, D), jnp.float32),
        pltpu.VMEM((BPW, D), jnp.float32),
        pltpu.VMEM((BPW,), jnp.float32),
        pltpu.VMEM((80,), jnp.float32),
        pltpu.SemaphoreType.DMA,
        pltpu.SemaphoreType.DMA,
    ],
)
def _mfnet_sc(uidx_hbm, iidx_hbm, u_hbm, v_hbm, wb_hbm, out_hbm,
              uidx_v, iidx_v, urows, vrows, outv, wbv, sem_u, sem_v):
    wid = lax.axis_index("s") * NC + lax.axis_index("c")
    base = wid * BPW

    pltpu.sync_copy(uidx_hbm.at[pl.ds(wid * NCHUNK, NCHUNK)], uidx_v)
    pltpu.sync_copy(iidx_hbm.at[pl.ds(wid * NCHUNK, NCHUNK)], iidx_v)
    pltpu.sync_copy(wb_hbm, wbv)

    copies = []
    for j in range(NCHUNK):
        copies.append(pltpu.async_copy(
            u_hbm.at[uidx_v.at[j]], urows.at[pl.ds(j * CHUNK, CHUNK)], sem_u))
        copies.append(pltpu.async_copy(
            v_hbm.at[iidx_v.at[j]], vrows.at[pl.ds(j * CHUNK, CHUNK)], sem_v))

    w0 = wbv[pl.ds(0, L)]
    w1 = wbv[pl.ds(L, L)]
    w2 = wbv[pl.ds(2 * L, L)]
    w3 = wbv[pl.ds(3 * L, L)]
    bias = wbv[pl.ds(D, L)][0]

    for c in copies:
        c.wait()

    lanes = lax.iota(jnp.int32, L)
    perms = [jnp.bitwise_xor(lanes, sh)[:, None] for sh in (8, 4, 2, 1)]
    gdn = lax.GatherDimensionNumbers(
        offset_dims=(), collapsed_slice_dims=(0,), start_index_map=(0,))

    def lane_sum(p):
        for perm in perms:
            p = p + lax.gather(
                p, perm, gdn, (1,),
                mode=lax.GatherScatterMode.PROMISE_IN_BOUNDS)
        return p

    def group_body(g, carry):
        r0 = g * L
        res = jnp.zeros((L,), jnp.float32)
        for k in range(L):
            r = r0 + k
            p = (urows[r, pl.ds(0, L)] * vrows[r, pl.ds(0, L)] * w0
                 + urows[r, pl.ds(L, L)] * vrows[r, pl.ds(L, L)] * w1
                 + urows[r, pl.ds(2 * L, L)] * vrows[r, pl.ds(2 * L, L)] * w2
                 + urows[r, pl.ds(3 * L, L)] * vrows[r, pl.ds(3 * L, L)] * w3)
            res = jnp.where(lanes == k, lane_sum(p), res)
        outv[pl.ds(r0, L)] = res + bias
        return carry

    lax.fori_loop(0, BPW // L, group_body, 0)

    pltpu.sync_copy(outv, out_hbm.at[pl.ds(base, BPW)])


def kernel(user_idx, item_idx, U, V, W, b):
    ui = user_idx.astype(jnp.int32).reshape(NW * NCHUNK, CHUNK)
    ii = item_idx.astype(jnp.int32).reshape(NW * NCHUNK, CHUNK)
    wb = jnp.concatenate(
        [W.reshape(-1), b.reshape(-1), jnp.zeros((80 - D - 1,), jnp.float32)])
    out = _mfnet_sc(ui, ii, U, V, wb)
    return out.reshape(B, 1)


# speedup vs baseline: 2.5385x; 2.5385x over previous
import functools

import jax
import jax.numpy as jnp
from jax import lax
from jax.experimental import pallas as pl
from jax.experimental.pallas import tpu as pltpu
from jax.experimental.pallas import tpu_sc as plsc

NC = 2
NS = 16
NW = NC * NS
L = 16

B = 16384
D = 64
BPW = B // NW
NBANK = 4

_mesh = plsc.VectorSubcoreMesh(core_axis_name="c", subcore_axis_name="s")


@functools.partial(
    pl.kernel,
    out_type=jax.ShapeDtypeStruct((B,), jnp.float32),
    mesh=_mesh,
    scratch_types=[
        pltpu.VMEM((BPW + L,), jnp.int32),
        pltpu.VMEM((BPW + L,), jnp.int32),
        [pltpu.VMEM((D, 128), jnp.float32)] * NBANK,
        [pltpu.VMEM((D, 128), jnp.float32)] * NBANK,
        pltpu.VMEM((BPW,), jnp.float32),
        pltpu.VMEM((80,), jnp.float32),
        [pltpu.SemaphoreType.DMA] * NBANK,
        [pltpu.SemaphoreType.DMA] * NBANK,
    ],
)
def _mfnet_sc(uidx_hbm, iidx_hbm, ut_hbm, vt_hbm, wb_hbm, out_hbm,
              uidx_v, iidx_v, ublk, vblk, outv, wbv, sems_u, sems_v):
    wid = lax.axis_index("s") * NC + lax.axis_index("c")
    base = wid * BPW

    pltpu.sync_copy(uidx_hbm.at[pl.ds(base, BPW)], uidx_v.at[pl.ds(0, BPW)])
    pltpu.sync_copy(iidx_hbm.at[pl.ds(base, BPW)], iidx_v.at[pl.ds(0, BPW)])
    pltpu.sync_copy(wb_hbm, wbv)

    w = [wbv[pl.ds(a * L, L)] for a in range(4)]
    bias = wbv[pl.ds(D, L)][0]

    lanes = lax.iota(jnp.int32, L)
    perms = [jnp.bitwise_xor(lanes, sh)[:, None] for sh in (8, 4, 2, 1)]
    gdn = lax.GatherDimensionNumbers(
        offset_dims=(), collapsed_slice_dims=(0,), start_index_map=(0,))

    def lane_sum(p):
        for perm in perms:
            p = p + lax.gather(
                p, perm, gdn, (1,),
                mode=lax.GatherScatterMode.PROMISE_IN_BOUNDS)
        return p

    def row_scalar(idx_ref, r):
        return idx_ref[pl.ds(r, L)][0]

    def issue(idx_ref, tbl, blk, sem, r):
        q = lax.shift_right_logical(row_scalar(idx_ref, r), 7)
        off = pl.multiple_of(q * 128, 128)
        pltpu.async_copy(tbl.at[:, pl.ds(off, 128)], blk, sem)

    for j in range(NBANK):
        issue(uidx_v, ut_hbm, ublk[j], sems_u[j], j)
        issue(iidx_v, vt_hbm, vblk[j], sems_v[j], j)

    zeros = jnp.zeros((L,), jnp.float32)

    def quad_body(qq, res):
        for j in range(NBANK):
            r = qq * NBANK + j
            pltpu.make_async_copy(
                ut_hbm.at[:, pl.ds(0, 128)], ublk[j], sems_u[j]).wait()
            pltpu.make_async_copy(
                vt_hbm.at[:, pl.ds(0, 128)], vblk[j], sems_v[j]).wait()

            cu = jnp.bitwise_and(row_scalar(uidx_v, r), 127)
            ci = jnp.bitwise_and(row_scalar(iidx_v, r), 127)
            msl_u = jnp.bitwise_and(cu, ~15)
            msl_i = jnp.bitwise_and(ci, ~15)
            ml_u = jnp.full((L,), jnp.bitwise_and(cu, 15), jnp.int32)[:, None]
            ml_i = jnp.full((L,), jnp.bitwise_and(ci, 15), jnp.int32)[:, None]
            p = zeros
            for a in range(4):
                ua = zeros
                va = zeros
                for k in range(L):
                    xu = ublk[j][a * L + k, pl.ds(msl_u, L)]
                    yu = lax.gather(xu, ml_u, gdn, (1,),
                                    mode=lax.GatherScatterMode.PROMISE_IN_BOUNDS)
                    ua = jnp.where(lanes == k, yu, ua)
                    xv = vblk[j][a * L + k, pl.ds(msl_i, L)]
                    yv = lax.gather(xv, ml_i, gdn, (1,),
                                    mode=lax.GatherScatterMode.PROMISE_IN_BOUNDS)
                    va = jnp.where(lanes == k, yv, va)
                p = p + ua * va * w[a]
            res = jnp.where(lanes == jnp.bitwise_and(r, 15), lane_sum(p), res)

            rn = r + NBANK
            @pl.when(rn < BPW)
            def _():
                issue(uidx_v, ut_hbm, ublk[j], sems_u[j], rn)
                issue(iidx_v, vt_hbm, vblk[j], sems_v[j], rn)

            if j == NBANK - 1:
                store = jnp.bitwise_and(r, 15) == 15

                @pl.when(store)
                def _():
                    outv[pl.ds(r - 15, L)] = res + bias
                res = jnp.where(store, zeros, res)
        return res

    lax.fori_loop(0, BPW // NBANK, quad_body, zeros)

    pltpu.sync_copy(outv, out_hbm.at[pl.ds(base, BPW)])


def kernel(user_idx, item_idx, U, V, W, b):
    ui = user_idx.astype(jnp.int32)
    ii = item_idx.astype(jnp.int32)
    wb = jnp.concatenate(
        [W.reshape(-1), b.reshape(-1), jnp.zeros((80 - D - 1,), jnp.float32)])
    out = _mfnet_sc(ui, ii, U.T, V.T, wb)
    return out.reshape(B, 1)
